# baseline (speedup 1.0000x reference)
import jax
import jax.numpy as jnp
from jax import lax
from jax.experimental import pallas as pl
from jax.experimental.pallas import tpu as pltpu
from jax.experimental.pallas import tpu_sc as plsc

_TOTAL = 1_000_000
_HIDDEN = 64
_WINDOW = 128
_BATCH = 4096
_L = 16
_NC, _NS = 2, 16
_NW = _NC * _NS
_BPW = _BATCH // _NW
_NBUF = 4
_NGRP = _BPW // _NBUF
_FETCH = 2 * _WINDOW
_WBUF = _FETCH + _L
_NCHUNK = 9
_TAILW = 2 * _WINDOW
_TAILBASE = _TOTAL - _TAILW
_MAINMAX = ((_TOTAL - _FETCH) // _WINDOW) * _WINDOW


def _sc_body(mu_hbm, sigma_hbm, table_hbm, tail_hbm, out_hbm, start_hbm,
             mu_v, sig_v, start_v, sal_v, center_v, nd_v, t_v, res_v,
             win0, win1, win2, win3, sem0, sem1, sem2, sem3):
    wins = [win0, win1, win2, win3]
    sems = [sem0, sem1, sem2, sem3]

    wid = lax.axis_index("s") * _NC + lax.axis_index("c")
    base = wid * _BPW

    pltpu.sync_copy(mu_hbm.at[pl.ds(base, _BPW)], mu_v)
    pltpu.sync_copy(sigma_hbm.at[pl.ds(base, _BPW)], sig_v)

    iota_i = lax.iota(jnp.int32, _L)
    zeros = jnp.zeros((_L,), jnp.float32)

    for c in range(_BPW // _L):
        sl = pl.ds(c * _L, _L)
        m = mu_v[sl]
        center = m * jnp.float32(_TOTAL - 1)
        sf = jnp.clip(center - jnp.float32(_WINDOW // 2),
                      jnp.float32(0.0), jnp.float32(_TOTAL - _WINDOW))
        si = sf.astype(jnp.int32)
        start_v[sl] = si
        salm = si & jnp.int32(~127)
        sal = jnp.where(salm > jnp.int32(_MAINMAX), jnp.int32(_TAILBASE), salm)
        sal_v[sl] = sal
        center_v[sl] = center
        sgp = sig_v[sl] + jnp.float32(1e-6)
        nd_v[sl] = jnp.float32(-0.5) / (sgp * sgp)

    pltpu.sync_copy(start_v.at[pl.ds(0, _BPW)], start_hbm.at[pl.ds(base, _BPW)])

    def issue(b, r):
        sal = sal_v[pl.ds(b, _L)][0]
        is_tail = sal == jnp.int32(_TAILBASE)

        @pl.when(jnp.logical_not(is_tail))
        def _():
            s_al = pl.multiple_of(sal, 128)
            pltpu.async_copy(table_hbm.at[:, pl.ds(s_al, _FETCH)],
                             wins[r].at[:, pl.ds(0, _FETCH)], sems[r])

        @pl.when(is_tail)
        def _():
            pltpu.async_copy(tail_hbm, wins[r].at[:, pl.ds(0, _FETCH)],
                             sems[r])

    for r in range(_NBUF):
        issue(r, r)

    for w in wins:
        for d in range(_HIDDEN):
            w[d, pl.ds(_WBUF - _L, _L)] = zeros

    def do_window(b, r):
        pltpu.make_async_copy(tail_hbm, wins[r].at[:, pl.ds(0, _FETCH)],
                              sems[r]).wait()
        sal = sal_v[pl.ds(b, _L)][0]
        si = start_v[pl.ds(b, _L)][0]
        ce = center_v[pl.ds(b, _L)][0]
        nd = nd_v[pl.ds(b, _L)][0]
        coff = (si - sal) & jnp.int32(~15)

        pb_b = jnp.broadcast_to(sal + coff, (_L,))

        ws = []
        ssum = None
        for c in range(_NCHUNK):
            pi = pb_b + (iota_i + jnp.int32(c * _L))
            d = pi.astype(jnp.float32) - ce
            w = jnp.exp(d * d * nd)
            if c == _NCHUNK - 1:
                w = jnp.where(pi < jnp.broadcast_to(si + jnp.int32(_WINDOW),
                                                    (_L,)),
                              w, jnp.float32(0.0))
            ws.append(w)
            ssum = w if ssum is None else ssum + w
        wsum = jnp.sum(ssum)
        rv = jnp.float32(1.0) / (jnp.broadcast_to(wsum, (_L,)) +
                                 jnp.float32(1e-6))

        win = wins[r]

        @plsc.parallel_loop(0, _HIDDEN, unroll=8)
        def _(d):
            p = [ws[c] * win[d, pl.ds(coff + c * _L, _L)]
                 for c in range(_NCHUNK)]
            t = ((((p[0] + p[1]) + (p[2] + p[3])) +
                  ((p[4] + p[5]) + (p[6] + p[7]))) + p[8])
            t_v[d, :] = t

        for k in range(_HIDDEN // _L):
            rows = iota_i + jnp.int32(k * _L)
            acc = None
            for l in range(_L):
                col = jnp.broadcast_to(jnp.int32(l), (_L,))
                g = plsc.load_gather(t_v, [rows, col])
                acc = g if acc is None else acc + g
            res_v[b, pl.ds(k * _L, _L)] = acc * rv

    def group(g, carry):
        for r in range(_NBUF):
            b = g * _NBUF + r
            do_window(b, r)

            @pl.when(g < _NGRP - 1)
            def _():
                issue(b + _NBUF, r)

        return carry

    lax.fori_loop(0, _NGRP, group, 0)

    pltpu.sync_copy(res_v, out_hbm.at[pl.ds(base, _BPW)])


@jax.jit
def _run(mu, sigma, params_t, tail):
    mesh = plsc.VectorSubcoreMesh(core_axis_name="c", subcore_axis_name="s",
                                  num_cores=_NC, num_subcores=_NS)
    f = pl.kernel(
        _sc_body,
        out_type=(
            jax.ShapeDtypeStruct((_BATCH, _HIDDEN), jnp.float32),
            jax.ShapeDtypeStruct((_BATCH,), jnp.int32),
        ),
        mesh=mesh,
        compiler_params=pltpu.CompilerParams(needs_layout_passes=False),
        scratch_types=[
            pltpu.VMEM((_BPW,), jnp.float32),
            pltpu.VMEM((_BPW,), jnp.float32),
            pltpu.VMEM((_BPW + _L,), jnp.int32),
            pltpu.VMEM((_BPW + _L,), jnp.int32),
            pltpu.VMEM((_BPW + _L,), jnp.float32),
            pltpu.VMEM((_BPW + _L,), jnp.float32),
            pltpu.VMEM((_HIDDEN, _L), jnp.float32),
            pltpu.VMEM((_BPW, _HIDDEN), jnp.float32),
            pltpu.VMEM((_HIDDEN, _WBUF), jnp.float32),
            pltpu.VMEM((_HIDDEN, _WBUF), jnp.float32),
            pltpu.VMEM((_HIDDEN, _WBUF), jnp.float32),
            pltpu.VMEM((_HIDDEN, _WBUF), jnp.float32),
            pltpu.SemaphoreType.DMA,
            pltpu.SemaphoreType.DMA,
            pltpu.SemaphoreType.DMA,
            pltpu.SemaphoreType.DMA,
        ],
    )
    return f(mu, sigma, params_t, tail)


def kernel(mu, sigma, params_storage):
    pt = params_storage.T
    tail = lax.slice(pt, (0, _TAILBASE), (_HIDDEN, _TOTAL))
    return _run(mu, sigma, pt, tail)


# speedup vs baseline: 1.0738x; 1.0738x over previous
import jax
import jax.numpy as jnp
from jax import lax
from jax.experimental import pallas as pl
from jax.experimental.pallas import tpu as pltpu
from jax.experimental.pallas import tpu_sc as plsc

_TOTAL = 1_000_000
_HIDDEN = 64
_WINDOW = 128
_BATCH = 4096
_L = 16
_NC, _NS = 2, 16
_NW = _NC * _NS
_BPW = _BATCH // _NW
_NBUF = 4
_NGRP = _BPW // _NBUF
_FETCH = 2 * _WINDOW
_WBUF = _FETCH + _L
_NCHUNK = 9
_TAILW = 2 * _WINDOW
_TAILBASE = _TOTAL - _TAILW
_MAINMAX = ((_TOTAL - _FETCH) // _WINDOW) * _WINDOW


def _sc_body(mu_hbm, sigma_hbm, table_hbm, tail_hbm, out_hbm, start_hbm,
             mu_v, sig_v, start_v, sal_v, center_v, nd_v, t_v, res_v,
             win0, win1, win2, win3, sem0, sem1, sem2, sem3):
    wins = [win0, win1, win2, win3]
    sems = [sem0, sem1, sem2, sem3]

    wid = lax.axis_index("s") * _NC + lax.axis_index("c")
    base = wid * _BPW

    pltpu.sync_copy(mu_hbm.at[pl.ds(base, _BPW)], mu_v)
    pltpu.sync_copy(sigma_hbm.at[pl.ds(base, _BPW)], sig_v)

    iota_i = lax.iota(jnp.int32, _L)
    zeros = jnp.zeros((_L,), jnp.float32)

    for c in range(_BPW // _L):
        sl = pl.ds(c * _L, _L)
        m = mu_v[sl]
        center = m * jnp.float32(_TOTAL - 1)
        sf = jnp.clip(center - jnp.float32(_WINDOW // 2),
                      jnp.float32(0.0), jnp.float32(_TOTAL - _WINDOW))
        si = sf.astype(jnp.int32)
        start_v[sl] = si
        salm = si & jnp.int32(~127)
        sal = jnp.where(salm > jnp.int32(_MAINMAX), jnp.int32(_TAILBASE), salm)
        sal_v[sl] = sal
        center_v[sl] = center
        sgp = sig_v[sl] + jnp.float32(1e-6)
        nd_v[sl] = jnp.float32(-0.5) / (sgp * sgp)

    pltpu.sync_copy(start_v.at[pl.ds(0, _BPW)], start_hbm.at[pl.ds(base, _BPW)])

    def issue(b, r):
        sal = sal_v[pl.ds(b, _L)][0]
        is_tail = sal == jnp.int32(_TAILBASE)

        @pl.when(jnp.logical_not(is_tail))
        def _():
            s_al = pl.multiple_of(sal, 128)
            pltpu.async_copy(table_hbm.at[:, pl.ds(s_al, _FETCH)],
                             wins[r].at[:, pl.ds(0, _FETCH)], sems[r])

        @pl.when(is_tail)
        def _():
            pltpu.async_copy(tail_hbm, wins[r].at[:, pl.ds(0, _FETCH)],
                             sems[r])

    for r in range(_NBUF):
        issue(r, r)

    for w in wins:
        for d in range(_HIDDEN):
            w[d, pl.ds(_WBUF - _L, _L)] = zeros

    def do_window(b, r):
        pltpu.make_async_copy(tail_hbm, wins[r].at[:, pl.ds(0, _FETCH)],
                              sems[r]).wait()
        sal = sal_v[pl.ds(b, _L)][0]
        si = start_v[pl.ds(b, _L)][0]
        ce = center_v[pl.ds(b, _L)][0]
        nd = nd_v[pl.ds(b, _L)][0]
        coff = (si - sal) & jnp.int32(~15)

        pb_b = jnp.broadcast_to(sal + coff, (_L,))

        ws = []
        ssum = None
        for c in range(_NCHUNK):
            pi = pb_b + (iota_i + jnp.int32(c * _L))
            d = pi.astype(jnp.float32) - ce
            w = jnp.exp(d * d * nd)
            if c == _NCHUNK - 1:
                w = jnp.where(pi < jnp.broadcast_to(si + jnp.int32(_WINDOW),
                                                    (_L,)),
                              w, jnp.float32(0.0))
            ws.append(w)
            ssum = w if ssum is None else ssum + w
        wsum = jnp.sum(ssum)
        rv = jnp.float32(1.0) / (jnp.broadcast_to(wsum, (_L,)) +
                                 jnp.float32(1e-6))

        win = wins[r]

        @plsc.parallel_loop(0, _HIDDEN, unroll=2)
        def _(d):
            p = [ws[c] * win[d, pl.ds(coff + c * _L, _L)]
                 for c in range(_NCHUNK)]
            t = ((((p[0] + p[1]) + (p[2] + p[3])) +
                  ((p[4] + p[5]) + (p[6] + p[7]))) + p[8])
            t_v[d, :] = t

        for k in range(_HIDDEN // _L):
            rows = iota_i + jnp.int32(k * _L)
            acc = None
            for l in range(_L):
                col = jnp.broadcast_to(jnp.int32(l), (_L,))
                g = plsc.load_gather(t_v, [rows, col])
                acc = g if acc is None else acc + g
            res_v[b, pl.ds(k * _L, _L)] = acc * rv

    def group(g, carry):
        for r in range(_NBUF):
            b = g * _NBUF + r
            do_window(b, r)

            @pl.when(g < _NGRP - 1)
            def _():
                issue(b + _NBUF, r)

        return carry

    lax.fori_loop(0, _NGRP, group, 0)

    pltpu.sync_copy(res_v, out_hbm.at[pl.ds(base, _BPW)])


@jax.jit
def _run(mu, sigma, params_t, tail):
    mesh = plsc.VectorSubcoreMesh(core_axis_name="c", subcore_axis_name="s",
                                  num_cores=_NC, num_subcores=_NS)
    f = pl.kernel(
        _sc_body,
        out_type=(
            jax.ShapeDtypeStruct((_BATCH, _HIDDEN), jnp.float32),
            jax.ShapeDtypeStruct((_BATCH,), jnp.int32),
        ),
        mesh=mesh,
        compiler_params=pltpu.CompilerParams(needs_layout_passes=False),
        scratch_types=[
            pltpu.VMEM((_BPW,), jnp.float32),
            pltpu.VMEM((_BPW,), jnp.float32),
            pltpu.VMEM((_BPW + _L,), jnp.int32),
            pltpu.VMEM((_BPW + _L,), jnp.int32),
            pltpu.VMEM((_BPW + _L,), jnp.float32),
            pltpu.VMEM((_BPW + _L,), jnp.float32),
            pltpu.VMEM((_HIDDEN, _L), jnp.float32),
            pltpu.VMEM((_BPW, _HIDDEN), jnp.float32),
            pltpu.VMEM((_HIDDEN, _WBUF), jnp.float32),
            pltpu.VMEM((_HIDDEN, _WBUF), jnp.float32),
            pltpu.VMEM((_HIDDEN, _WBUF), jnp.float32),
            pltpu.VMEM((_HIDDEN, _WBUF), jnp.float32),
            pltpu.SemaphoreType.DMA,
            pltpu.SemaphoreType.DMA,
            pltpu.SemaphoreType.DMA,
            pltpu.SemaphoreType.DMA,
        ],
    )
    return f(mu, sigma, params_t, tail)


def kernel(mu, sigma, params_storage):
    pt = params_storage.T
    tail = lax.slice(pt, (0, _TAILBASE), (_HIDDEN, _TOTAL))
    return _run(mu, sigma, pt, tail)


# speedup vs baseline: 1.1331x; 1.0552x over previous
import jax
import jax.numpy as jnp
from jax import lax
from jax.experimental import pallas as pl
from jax.experimental.pallas import tpu as pltpu
from jax.experimental.pallas import tpu_sc as plsc

_TOTAL = 1_000_000
_HIDDEN = 64
_WINDOW = 128
_BATCH = 4096
_L = 16
_NC, _NS = 2, 16
_NW = _NC * _NS
_BPW = _BATCH // _NW
_NBUF = 4
_NGRP = _BPW // _NBUF
_FETCH = 2 * _WINDOW
_WBUF = _FETCH + _L
_NCHUNK = 9
_TAILW = 2 * _WINDOW
_TAILBASE = _TOTAL - _TAILW
_MAINMAX = ((_TOTAL - _FETCH) // _WINDOW) * _WINDOW


def _sc_body(mu_hbm, sigma_hbm, table_hbm, tail_hbm, out_hbm, start_hbm,
             mu_v, sig_v, start_v, sal_v, center_v, nd_v, t_v, res_v,
             win0, win1, win2, win3, sem0, sem1, sem2, sem3):
    wins = [win0, win1, win2, win3]
    sems = [sem0, sem1, sem2, sem3]

    wid = lax.axis_index("s") * _NC + lax.axis_index("c")
    base = wid * _BPW

    pltpu.sync_copy(mu_hbm.at[pl.ds(base, _BPW)], mu_v)
    pltpu.sync_copy(sigma_hbm.at[pl.ds(base, _BPW)], sig_v)

    iota_i = lax.iota(jnp.int32, _L)
    zeros = jnp.zeros((_L,), jnp.float32)

    for c in range(_BPW // _L):
        sl = pl.ds(c * _L, _L)
        m = mu_v[sl]
        center = m * jnp.float32(_TOTAL - 1)
        sf = jnp.clip(center - jnp.float32(_WINDOW // 2),
                      jnp.float32(0.0), jnp.float32(_TOTAL - _WINDOW))
        si = sf.astype(jnp.int32)
        start_v[sl] = si
        salm = si & jnp.int32(~127)
        sal = jnp.where(salm > jnp.int32(_MAINMAX), jnp.int32(_TAILBASE), salm)
        sal_v[sl] = sal
        center_v[sl] = center
        sgp = sig_v[sl] + jnp.float32(1e-6)
        nd_v[sl] = jnp.float32(-0.5) / (sgp * sgp)

    pltpu.sync_copy(start_v.at[pl.ds(0, _BPW)], start_hbm.at[pl.ds(base, _BPW)])

    def issue(b, r):
        sal = sal_v[pl.ds(b, _L)][0]
        is_tail = sal == jnp.int32(_TAILBASE)

        @pl.when(jnp.logical_not(is_tail))
        def _():
            s_al = pl.multiple_of(sal, 128)
            pltpu.async_copy(table_hbm.at[:, pl.ds(s_al, _FETCH)],
                             wins[r].at[:, pl.ds(0, _FETCH)], sems[r])

        @pl.when(is_tail)
        def _():
            pltpu.async_copy(tail_hbm, wins[r].at[:, pl.ds(0, _FETCH)],
                             sems[r])

    for r in range(_NBUF):
        issue(r, r)

    for w in wins:
        for d in range(_HIDDEN):
            w[d, pl.ds(_WBUF - _L, _L)] = zeros

    def do_window(b, r):
        pltpu.make_async_copy(tail_hbm, wins[r].at[:, pl.ds(0, _FETCH)],
                              sems[r]).wait()
        sal = sal_v[pl.ds(b, _L)][0]
        si = start_v[pl.ds(b, _L)][0]
        ce = center_v[pl.ds(b, _L)][0]
        nd = nd_v[pl.ds(b, _L)][0]
        coff = (si - sal) & jnp.int32(~15)

        pb_b = jnp.broadcast_to(sal + coff, (_L,))

        ws = []
        ssum = None
        for c in range(_NCHUNK):
            pi = pb_b + (iota_i + jnp.int32(c * _L))
            d = pi.astype(jnp.float32) - ce
            w = jnp.exp(d * d * nd)
            if c == _NCHUNK - 1:
                w = jnp.where(pi < jnp.broadcast_to(si + jnp.int32(_WINDOW),
                                                    (_L,)),
                              w, jnp.float32(0.0))
            ws.append(w)
            ssum = w if ssum is None else ssum + w
        wsum = jnp.sum(ssum)
        rv = jnp.float32(1.0) / (jnp.broadcast_to(wsum, (_L,)) +
                                 jnp.float32(1e-6))

        win = wins[r]

        @plsc.parallel_loop(0, _HIDDEN, unroll=4)
        def _(d):
            p = [ws[c] * win[d, pl.ds(coff + c * _L, _L)]
                 for c in range(_NCHUNK)]
            t = ((((p[0] + p[1]) + (p[2] + p[3])) +
                  ((p[4] + p[5]) + (p[6] + p[7]))) + p[8])
            t_v[d, :] = t

        @plsc.parallel_loop(0, _HIDDEN // _L, unroll=4)
        def _(k):
            rows = iota_i + k * _L
            g = [plsc.load_gather(t_v,
                                  [rows, jnp.broadcast_to(jnp.int32(l), (_L,))])
                 for l in range(_L)]
            s0 = (g[0] + g[1]) + (g[2] + g[3])
            s1 = (g[4] + g[5]) + (g[6] + g[7])
            s2 = (g[8] + g[9]) + (g[10] + g[11])
            s3 = (g[12] + g[13]) + (g[14] + g[15])
            acc = (s0 + s1) + (s2 + s3)
            res_v[b, pl.ds(k * _L, _L)] = acc * rv

    def group(g, carry):
        for r in range(_NBUF):
            b = g * _NBUF + r
            do_window(b, r)

            @pl.when(g < _NGRP - 1)
            def _():
                issue(b + _NBUF, r)

        return carry

    lax.fori_loop(0, _NGRP, group, 0)

    pltpu.sync_copy(res_v, out_hbm.at[pl.ds(base, _BPW)])


@jax.jit
def _run(mu, sigma, params_t, tail):
    mesh = plsc.VectorSubcoreMesh(core_axis_name="c", subcore_axis_name="s",
                                  num_cores=_NC, num_subcores=_NS)
    f = pl.kernel(
        _sc_body,
        out_type=(
            jax.ShapeDtypeStruct((_BATCH, _HIDDEN), jnp.float32),
            jax.ShapeDtypeStruct((_BATCH,), jnp.int32),
        ),
        mesh=mesh,
        compiler_params=pltpu.CompilerParams(needs_layout_passes=False),
        scratch_types=[
            pltpu.VMEM((_BPW,), jnp.float32),
            pltpu.VMEM((_BPW,), jnp.float32),
            pltpu.VMEM((_BPW + _L,), jnp.int32),
            pltpu.VMEM((_BPW + _L,), jnp.int32),
            pltpu.VMEM((_BPW + _L,), jnp.float32),
            pltpu.VMEM((_BPW + _L,), jnp.float32),
            pltpu.VMEM((_HIDDEN, _L), jnp.float32),
            pltpu.VMEM((_BPW, _HIDDEN), jnp.float32),
            pltpu.VMEM((_HIDDEN, _WBUF), jnp.float32),
            pltpu.VMEM((_HIDDEN, _WBUF), jnp.float32),
            pltpu.VMEM((_HIDDEN, _WBUF), jnp.float32),
            pltpu.VMEM((_HIDDEN, _WBUF), jnp.float32),
            pltpu.SemaphoreType.DMA,
            pltpu.SemaphoreType.DMA,
            pltpu.SemaphoreType.DMA,
            pltpu.SemaphoreType.DMA,
        ],
    )
    return f(mu, sigma, params_t, tail)


def kernel(mu, sigma, params_storage):
    pt = params_storage.T
    tail = lax.slice(pt, (0, _TAILBASE), (_HIDDEN, _TOTAL))
    return _run(mu, sigma, pt, tail)
